# baseline (speedup 1.0000x reference)
import functools

import jax
import jax.numpy as jnp
from jax import lax
from jax.experimental import pallas as pl
from jax.experimental.pallas import tpu as pltpu
from jax.experimental.pallas import tpu_sc as plsc

_BATCH = 4096
_SEQ = 200
_DT = 64
_DE = 32
_DOUT = _DT - 1 + _DE
_IDXC = 63
_NC = 2
_NS = 16
_NW = _NC * _NS
_BN = 128
_SB = 8
_GR = 64
_NG = _SB * _BN // _GR
_NRING = 2
_CPS = _BATCH // _BN
_NSLAB = (_SEQ // _SB) * _CPS
_NCH = _NSLAB // _NW
_G2S = 145


def _sc_body(tokens_hbm, table_hbm, out_hbm, tok_v, emb_v, idx_v,
             g0_v, g1_v, t_v, sem_rd, sem_wa, sem0, sem1):
    wid = lax.axis_index("s") * _NC + lax.axis_index("c")
    bufs = (g0_v, g1_v)
    sems = (sem0, sem1)
    iota_lo = lax.iota(jnp.int32, 16)
    iota_hi = iota_lo + 16

    def slab(i, carry):
        g = wid * _NCH + i
        s0 = (g // _CPS) * _SB
        b0 = (g % _CPS) * _BN

        copies = []
        for ss in range(_SB):
            copies.append(pltpu.async_copy(
                tokens_hbm.at[s0 + ss, :, pl.ds(b0, _BN)],
                tok_v.at[:, ss, :],
                sem_rd,
            ))
        for c in copies:
            c.wait()

        def ext_ss(ss, c):
            def ext_k(kk, c2):
                v = tok_v[_IDXC, ss, pl.ds(kk * 16, 16)]
                idx_v[ss, pl.ds(kk * 16, 16)] = v.astype(jnp.int32)
                return c2

            lax.fori_loop(0, _BN // 16, ext_k, 0, unroll=4)
            return c

        lax.fori_loop(0, _SB, ext_ss, 0)

        wa = pltpu.async_copy(
            tok_v.at[pl.ds(0, _IDXC), :, :],
            out_hbm.at[pl.ds(0, _IDXC), pl.ds(s0, _SB), pl.ds(b0, _BN)],
            sem_wa,
        )

        def issue(t, buf, sem):
            ss = t // 2
            h = t % 2
            return pltpu.async_copy(
                table_hbm.at[idx_v.at[ss, pl.ds(h * _GR, _GR)]], buf, sem
            )

        def stage1(t, buf):
            h = t % 2
            cbase = jnp.full((16,), h * _GR, jnp.int32)

            def rrow(r, c):
                a = buf[r, pl.ds(0, 16)]
                b = buf[r, pl.ds(16, 16)]
                col = cbase + r
                plsc.store_scatter(t_v, [iota_lo, col], a)
                plsc.store_scatter(t_v, [iota_hi, col], b)
                return c

            lax.fori_loop(0, _GR, rrow, 0, unroll=4)

        def stage2(ss):
            def jrow(j, c):
                def krow(kk, c2):
                    v = t_v[j, pl.ds(kk * 16, 16)]
                    emb_v[j, ss, pl.ds(kk * 16, 16)] = v
                    return c2

                lax.fori_loop(0, _BN // 16, krow, 0, unroll=4)
                return c

            lax.fori_loop(0, _DE, jrow, 0)

        descs = {}
        for t in range(_NRING):
            descs[t] = issue(t, bufs[t % _NRING], sems[t % _NRING])
        for t in range(_NG):
            descs.pop(t).wait()
            stage1(t, bufs[t % _NRING])
            if t + _NRING < _NG:
                descs[t + _NRING] = issue(
                    t + _NRING, bufs[t % _NRING], sems[t % _NRING]
                )
            if t % 2 == 1:
                stage2(t // 2)

        pltpu.sync_copy(
            emb_v, out_hbm.at[pl.ds(_IDXC, _DE), pl.ds(s0, _SB), pl.ds(b0, _BN)]
        )
        wa.wait()
        return carry

    lax.fori_loop(0, _NCH, slab, 0)


_sc_call = functools.partial(
    pl.kernel,
    out_type=jax.ShapeDtypeStruct((_DOUT, _SEQ, _BATCH), jnp.float32),
    mesh=plsc.VectorSubcoreMesh(core_axis_name="c", subcore_axis_name="s"),
    compiler_params=pltpu.CompilerParams(needs_layout_passes=False),
    scratch_types=[
        pltpu.VMEM((_DT, _SB, _BN), jnp.float32),
        pltpu.VMEM((_DE, _SB, _BN), jnp.float32),
        pltpu.VMEM((_SB, _BN), jnp.int32),
        pltpu.VMEM((_GR, 128), jnp.float32),
        pltpu.VMEM((_GR, 128), jnp.float32),
        pltpu.VMEM((_DE, _G2S), jnp.float32),
        pltpu.SemaphoreType.DMA,
        pltpu.SemaphoreType.DMA,
        pltpu.SemaphoreType.DMA,
        pltpu.SemaphoreType.DMA,
    ],
)(_sc_body)


def kernel(tokens, table):
    tokens_p = jnp.transpose(tokens, (1, 2, 0))
    table128 = jnp.pad(table, ((0, 0), (0, 128 - _DE)))
    out_p = _sc_call(tokens_p, table128)
    return jnp.transpose(out_p, (2, 1, 0))


# speedup vs baseline: 1.2388x; 1.2388x over previous
import functools

import jax
import jax.numpy as jnp
from jax import lax
from jax.experimental import pallas as pl
from jax.experimental.pallas import tpu as pltpu
from jax.experimental.pallas import tpu_sc as plsc

_BATCH = 4096
_SEQ = 200
_DT = 64
_DE = 32
_DOUT = _DT - 1 + _DE
_IDXC = 63
_NC = 2
_NS = 16
_NW = _NC * _NS
_BN = 128
_SB = 8
_GR = 64
_NG = _SB * _BN // _GR
_NRING = 2
_CPS = _BATCH // _BN
_NSLAB = (_SEQ // _SB) * _CPS
_NCH = _NSLAB // _NW
_G2S = 145


def _sc_body(tokens_hbm, table_hbm, out_hbm, tok_v, emb_v, idx_v,
             g0_v, g1_v, sem_rd, sem_wa, sem0, sem1):
    wid = lax.axis_index("s") * _NC + lax.axis_index("c")
    bufs = (g0_v, g1_v)
    sems = (sem0, sem1)
    iota_lo = lax.iota(jnp.int32, 16)
    iota_hi = iota_lo + 16

    def slab(i, carry):
        g = wid * _NCH + i
        s0 = (g // _CPS) * _SB
        b0 = (g % _CPS) * _BN

        copies = []
        for ss in range(_SB):
            copies.append(pltpu.async_copy(
                tokens_hbm.at[s0 + ss, :, pl.ds(b0, _BN)],
                tok_v.at[:, ss, :],
                sem_rd,
            ))
        for c in copies:
            c.wait()

        def ext_ss(ss, c):
            def ext_k(kk, c2):
                v = tok_v[_IDXC, ss, pl.ds(kk * 16, 16)]
                idx_v[ss, pl.ds(kk * 16, 16)] = v.astype(jnp.int32)
                return c2

            lax.fori_loop(0, _BN // 16, ext_k, 0, unroll=4)
            return c

        lax.fori_loop(0, _SB, ext_ss, 0)

        wa = pltpu.async_copy(
            tok_v.at[pl.ds(0, _IDXC), :, :],
            out_hbm.at[pl.ds(0, _IDXC), pl.ds(s0, _SB), pl.ds(b0, _BN)],
            sem_wa,
        )

        def issue(t, buf, sem):
            ss = t // 2
            h = t % 2
            return pltpu.async_copy(
                table_hbm.at[idx_v.at[ss, pl.ds(h * _GR, _GR)]], buf, sem
            )

        def stage1(t, buf):
            ss = t // 2
            h = t % 2
            ssv = jnp.full((16,), ss, jnp.int32)
            cbase = jnp.full((16,), h * _GR, jnp.int32)

            def rrow(r, c):
                a = buf[r, pl.ds(0, 16)]
                b = buf[r, pl.ds(16, 16)]
                col = cbase + r
                plsc.store_scatter(emb_v, [iota_lo, ssv, col], a)
                plsc.store_scatter(emb_v, [iota_hi, ssv, col], b)
                return c

            lax.fori_loop(0, _GR, rrow, 0, unroll=4)

        descs = {}
        for t in range(_NRING):
            descs[t] = issue(t, bufs[t % _NRING], sems[t % _NRING])
        for t in range(_NG):
            descs.pop(t).wait()
            stage1(t, bufs[t % _NRING])
            if t + _NRING < _NG:
                descs[t + _NRING] = issue(
                    t + _NRING, bufs[t % _NRING], sems[t % _NRING]
                )

        pltpu.sync_copy(
            emb_v, out_hbm.at[pl.ds(_IDXC, _DE), pl.ds(s0, _SB), pl.ds(b0, _BN)]
        )
        wa.wait()
        return carry

    lax.fori_loop(0, _NCH, slab, 0)


_sc_call = functools.partial(
    pl.kernel,
    out_type=jax.ShapeDtypeStruct((_DOUT, _SEQ, _BATCH), jnp.float32),
    mesh=plsc.VectorSubcoreMesh(core_axis_name="c", subcore_axis_name="s"),
    compiler_params=pltpu.CompilerParams(needs_layout_passes=False),
    scratch_types=[
        pltpu.VMEM((_DT, _SB, _BN), jnp.float32),
        pltpu.VMEM((_DE, _SB, _BN), jnp.float32),
        pltpu.VMEM((_SB, _BN), jnp.int32),
        pltpu.VMEM((_GR, 128), jnp.float32),
        pltpu.VMEM((_GR, 128), jnp.float32),
        pltpu.SemaphoreType.DMA,
        pltpu.SemaphoreType.DMA,
        pltpu.SemaphoreType.DMA,
        pltpu.SemaphoreType.DMA,
    ],
)(_sc_body)


def kernel(tokens, table):
    tokens_p = jnp.transpose(tokens, (1, 2, 0))
    table128 = jnp.pad(table, ((0, 0), (0, 128 - _DE)))
    out_p = _sc_call(tokens_p, table128)
    return jnp.transpose(out_p, (2, 1, 0))


# speedup vs baseline: 1.2396x; 1.0007x over previous
import functools

import jax
import jax.numpy as jnp
from jax import lax
from jax.experimental import pallas as pl
from jax.experimental.pallas import tpu as pltpu
from jax.experimental.pallas import tpu_sc as plsc

_BATCH = 4096
_SEQ = 200
_DT = 64
_DE = 32
_DOUT = _DT - 1 + _DE
_IDXC = 63
_NC = 2
_NS = 16
_NW = _NC * _NS
_BN = 128
_SB = 8
_GR = 64
_NG = _SB * _BN // _GR
_NRING = 2
_CPS = _BATCH // _BN
_NSLAB = (_SEQ // _SB) * _CPS
_NCH = _NSLAB // _NW


def _sc_body(tokens_hbm, table_hbm, out_hbm, tok_v, emb_v, idx_v,
             g0_v, g1_v, sem_rd, sem_wa, sem0, sem1):
    wid = lax.axis_index("s") * _NC + lax.axis_index("c")
    bufs = (g0_v, g1_v)
    sems = (sem0, sem1)
    iota_lo = lax.iota(jnp.int32, 16)
    iota_hi = iota_lo + 16

    def slab(i, carry):
        g = wid * _NCH + i
        s0 = (g // _CPS) * _SB
        b0 = (g % _CPS) * _BN

        copies = []
        for ss in range(_SB):
            copies.append(pltpu.async_copy(
                tokens_hbm.at[s0 + ss, :, pl.ds(b0, _BN)],
                tok_v.at[:, ss, :],
                sem_rd,
            ))
        for c in copies:
            c.wait()

        def ext_ss(ss, c):
            def ext_k(kk, c2):
                v = tok_v[_IDXC, ss, pl.ds(kk * 16, 16)]
                idx_v[ss, pl.ds(kk * 16, 16)] = v.astype(jnp.int32)
                return c2

            lax.fori_loop(0, _BN // 16, ext_k, 0, unroll=4)
            return c

        lax.fori_loop(0, _SB, ext_ss, 0)

        wa = pltpu.async_copy(
            tok_v.at[pl.ds(0, _IDXC), :, :],
            out_hbm.at[pl.ds(0, _IDXC), pl.ds(s0, _SB), pl.ds(b0, _BN)],
            sem_wa,
        )

        def issue(t, buf, sem):
            ss = t // 2
            h = t % 2
            return pltpu.async_copy(
                table_hbm.at[idx_v.at[ss, pl.ds(h * _GR, _GR)]], buf, sem
            )

        def stage1(t, buf):
            ss = t // 2
            h = t % 2
            ssv = jnp.full((16,), ss, jnp.int32)
            cbase = jnp.full((16,), h * _GR, jnp.int32)

            def rrow(r, c):
                a = buf[r, pl.ds(0, 16)]
                b = buf[r, pl.ds(16, 16)]
                col = cbase + r
                plsc.store_scatter(emb_v, [iota_lo, ssv, col], a)
                plsc.store_scatter(emb_v, [iota_hi, ssv, col], b)
                return c

            lax.fori_loop(0, _GR, rrow, 0, unroll=4)

        descs = {}
        for t in range(_NRING):
            descs[t] = issue(t, bufs[t % _NRING], sems[t % _NRING])
        for t in range(_NG):
            descs.pop(t).wait()
            stage1(t, bufs[t % _NRING])
            if t + _NRING < _NG:
                descs[t + _NRING] = issue(
                    t + _NRING, bufs[t % _NRING], sems[t % _NRING]
                )

        pltpu.sync_copy(
            emb_v, out_hbm.at[pl.ds(_IDXC, _DE), pl.ds(s0, _SB), pl.ds(b0, _BN)]
        )
        wa.wait()
        return carry

    lax.fori_loop(0, _NCH, slab, 0)


_sc_call = functools.partial(
    pl.kernel,
    out_type=jax.ShapeDtypeStruct((_DOUT, _SEQ, _BATCH), jnp.float32),
    mesh=plsc.VectorSubcoreMesh(core_axis_name="c", subcore_axis_name="s"),
    compiler_params=pltpu.CompilerParams(needs_layout_passes=False),
    scratch_types=[
        pltpu.VMEM((_DT, _SB, _BN), jnp.float32),
        pltpu.VMEM((_DE, _SB, _BN), jnp.float32),
        pltpu.VMEM((_SB, _BN), jnp.int32),
        pltpu.VMEM((_GR, 128), jnp.float32),
        pltpu.VMEM((_GR, 128), jnp.float32),
        pltpu.SemaphoreType.DMA,
        pltpu.SemaphoreType.DMA,
        pltpu.SemaphoreType.DMA,
        pltpu.SemaphoreType.DMA,
    ],
)(_sc_body)


def kernel(tokens, table):
    tokens_p = jnp.transpose(tokens, (1, 2, 0))
    table128 = jnp.pad(table, ((0, 0), (0, 128 - _DE)))
    out_p = _sc_call(tokens_p, table128)
    return jnp.transpose(out_p, (2, 1, 0))
